# baseline (speedup 1.0000x reference)
import jax
import jax.numpy as jnp
from jax import lax
from jax.experimental import pallas as pl
from jax.experimental.pallas import tpu as pltpu
from jax.experimental.pallas import tpu_sc as plsc

D = 128
SCALE = float(D) ** 0.5
NC, NS = 2, 16
NW = NC * NS
NSPLIT = 2
XROWS = 4096
SEQ = 50
SEQP = 56
XSPL = XROWS // NSPLIT
RPW = XSPL // NW
CROWS = 1
NCHUNK = RPW // CROWS
NBUF = 4
PREF = 3


def _emb_body(x_hbm, table_hbm, out_hbm, idx_v,
              r0, r1, r2, r3,
              g0, g1, g2, g3,
              w0, w1, w2, w3):
    bufs = (r0, r1, r2, r3)
    gsems = (g0, g1, g2, g3)
    wsems = (w0, w1, w2, w3)

    wid = lax.axis_index("s") * NC + lax.axis_index("c")
    xbase = wid * RPW
    pltpu.sync_copy(x_hbm.at[pl.ds(xbase, RPW), :], idx_v)

    def gather(c, b):
        return pltpu.make_async_copy(
            table_hbm.at[idx_v.at[c]], bufs[b], gsems[b])

    def scatter(c, b):
        return pltpu.make_async_copy(
            bufs[b], out_hbm.at[xbase + c], wsems[b])

    for b in range(PREF):
        gather(b, b).start()

    def outer(g, carry):
        for b in range(NBUF):
            c = g * NBUF + b

            bp = (b + PREF) % NBUF

            @pl.when(c + PREF < NCHUNK)
            def _():
                @pl.when(c + PREF - NBUF >= 0)
                def _():
                    scatter(c + PREF - NBUF, bp).wait()
                gather(c + PREF, bp).start()

            gather(c, b).wait()

            def row_body(i, carry2):
                for j in range(D // 16):
                    sl = pl.ds(j * 16, 16)
                    bufs[b][i, sl] = bufs[b][i, sl] * SCALE
                return carry2

            lax.fori_loop(0, SEQ, row_body, 0, unroll=2)
            scatter(c, b).start()
        return carry

    lax.fori_loop(0, NCHUNK // NBUF, outer, 0)

    for b in range(NBUF):
        scatter(NCHUNK - NBUF + b, b).wait()


@jax.jit
def kernel(x, tok_embed_weight):
    idx = x.astype(jnp.int32)
    mesh = plsc.VectorSubcoreMesh(
        core_axis_name="c", subcore_axis_name="s",
        num_cores=NC, num_subcores=NS,
    )
    call = pl.kernel(
        _emb_body,
        out_type=jax.ShapeDtypeStruct((XSPL, SEQ, D), jnp.float32),
        mesh=mesh,
        scratch_types=(
            [pltpu.VMEM((RPW, SEQ), jnp.int32)]
            + [pltpu.VMEM((SEQ, D), jnp.float32) for _ in range(NBUF)]
            + [pltpu.SemaphoreType.DMA for _ in range(2 * NBUF)]
        ),
    )
    parts = [call(idx[i * XSPL:(i + 1) * XSPL], tok_embed_weight)
             for i in range(NSPLIT)]
    return jnp.concatenate(parts, axis=0)


# speedup vs baseline: 1.6323x; 1.6323x over previous
import jax
import jax.numpy as jnp
from jax import lax
from jax.experimental import pallas as pl
from jax.experimental.pallas import tpu as pltpu
from jax.experimental.pallas import tpu_sc as plsc

D = 128
SCALE = float(D) ** 0.5
NC, NS = 2, 16
NW = NC * NS
XROWS = 4096
SEQ = 50
SEQP = 56
RPW = XROWS // NW
CROWS = 1
NCHUNK = RPW // CROWS
NBUF = 8
PREF = 6


def _emb_body(x_hbm, table_hbm, out_hbm, idx_v,
              r0, r1, r2, r3, r4, r5, r6, r7,
              g0, g1, g2, g3, g4, g5, g6, g7,
              w0, w1, w2, w3, w4, w5, w6, w7):
    bufs = (r0, r1, r2, r3, r4, r5, r6, r7)
    gsems = (g0, g1, g2, g3, g4, g5, g6, g7)
    wsems = (w0, w1, w2, w3, w4, w5, w6, w7)

    wid = lax.axis_index("s") * NC + lax.axis_index("c")
    xbase = wid * RPW
    pltpu.sync_copy(x_hbm.at[pl.ds(xbase, RPW), :], idx_v)

    def gather(c, b):
        return pltpu.make_async_copy(
            table_hbm.at[idx_v.at[c]], bufs[b], gsems[b])

    def scatter(c, b):
        return pltpu.make_async_copy(
            bufs[b], out_hbm.at[xbase + c], wsems[b])

    for b in range(PREF):
        gather(b, b).start()

    def outer(g, carry):
        for b in range(NBUF):
            c = g * NBUF + b

            bp = (b + PREF) % NBUF

            @pl.when(c + PREF < NCHUNK)
            def _():
                @pl.when(c + PREF - NBUF >= 0)
                def _():
                    scatter(c + PREF - NBUF, bp).wait()
                gather(c + PREF, bp).start()

            gather(c, b).wait()

            def row_body(i, carry2):
                for j in range(D // 16):
                    sl = pl.ds(j * 16, 16)
                    bufs[b][i, sl] = bufs[b][i, sl] * SCALE
                return carry2

            lax.fori_loop(0, SEQ, row_body, 0, unroll=2)
            scatter(c, b).start()
        return carry

    lax.fori_loop(0, NCHUNK // NBUF, outer, 0)

    for b in range(NBUF):
        scatter(NCHUNK - NBUF + b, b).wait()


@jax.jit
def kernel(x, tok_embed_weight):
    idx = x.astype(jnp.int32)
    mesh = plsc.VectorSubcoreMesh(
        core_axis_name="c", subcore_axis_name="s",
        num_cores=NC, num_subcores=NS,
    )
    out = pl.kernel(
        _emb_body,
        out_type=jax.ShapeDtypeStruct((XROWS, SEQ, D), jnp.float32),
        mesh=mesh,
        scratch_types=(
            [pltpu.VMEM((RPW, SEQ), jnp.int32)]
            + [pltpu.VMEM((SEQ, D), jnp.float32) for _ in range(NBUF)]
            + [pltpu.SemaphoreType.DMA for _ in range(2 * NBUF)]
        ),
    )(idx, tok_embed_weight)
    return out
